# baseline (speedup 1.0000x reference)
import functools

import jax
import jax.numpy as jnp
from jax import lax
from jax.experimental import pallas as pl
from jax.experimental.pallas import tpu as pltpu
from jax.experimental.pallas import tpu_sc as plsc

_NUM_CORES = 2
_NUM_SUBCORES = 16
_NUM_WORKERS = _NUM_CORES * _NUM_SUBCORES
_LANES = 16
_CHUNK = 128


@jax.jit
def _lr_pooled_lookup(x3, table_flat, bias16):
    NW, NG, C = x3.shape
    per_tile = NG * C
    F = 26
    bpw = per_tile // F
    B = NW * bpw
    mesh = plsc.VectorSubcoreMesh(core_axis_name="c", subcore_axis_name="s")

    @functools.partial(
        pl.kernel,
        out_type=jax.ShapeDtypeStruct((B,), jnp.float32),
        mesh=mesh,
        compiler_params=pltpu.CompilerParams(needs_layout_passes=False),
        scratch_types=[
            pltpu.VMEM((NG, C), jnp.int32),
            pltpu.VMEM((NG, C), jnp.float32),
            pltpu.VMEM((_LANES,), jnp.float32),
            pltpu.VMEM((bpw,), jnp.float32),
            pltpu.SemaphoreType.DMA,
        ],
    )
    def k(x_hbm, tab_hbm, bias_hbm, out_hbm, xv, vals_v, bias_v, acc_v, gsem):
        wid = lax.axis_index("s") * _NUM_CORES + lax.axis_index("c")
        base = wid * bpw
        pltpu.sync_copy(x_hbm.at[wid], xv)
        pltpu.sync_copy(bias_hbm, bias_v)
        copies = [
            pltpu.async_copy(tab_hbm.at[xv.at[g]], vals_v.at[g], gsem)
            for g in range(NG)
        ]
        for c in copies:
            c.wait()
        bvec = bias_v[...]
        iota26 = lax.iota(jnp.int32, _LANES) * F
        for i in range(bpw // _LANES):
            pos = iota26 + (i * _LANES * F)
            acc = bvec
            for f in range(F):
                p = pos + f
                acc = acc + plsc.load_gather(
                    vals_v, [lax.shift_right_logical(p, 7), lax.bitwise_and(p, 127)]
                )
            acc_v[pl.ds(i * _LANES, _LANES)] = acc
        pltpu.sync_copy(acc_v, out_hbm.at[pl.ds(base, bpw)])

    return k(x3, table_flat, bias16)


def kernel(X, table, bias):
    B, F = X.shape
    per_tile = (B // _NUM_WORKERS) * F
    x3 = X.reshape(_NUM_WORKERS, per_tile // _CHUNK, _CHUNK)
    out = _lr_pooled_lookup(x3, table.reshape(-1), jnp.broadcast_to(bias, (_LANES,)))
    return out.reshape(B, 1)


# speedup vs baseline: 2.1693x; 2.1693x over previous
import functools

import jax
import jax.numpy as jnp
from jax import lax
from jax.experimental import pallas as pl
from jax.experimental.pallas import tpu as pltpu
from jax.experimental.pallas import tpu_sc as plsc

_NUM_CORES = 2
_NUM_SUBCORES = 16
_NUM_WORKERS = _NUM_CORES * _NUM_SUBCORES
_LANES = 16
_CHUNK = 128


@jax.jit
def _lr_pooled_lookup(xt, table, bias16):
    F, NW, NJ, _one, C = xt.shape
    bpw = NJ * C
    B = NW * bpw
    mesh = plsc.VectorSubcoreMesh(core_axis_name="c", subcore_axis_name="s")

    @functools.partial(
        pl.kernel,
        out_type=jax.ShapeDtypeStruct((B,), jnp.float32),
        mesh=mesh,
        scratch_types=[
            pltpu.VMEM((F, NJ, 1, C), jnp.int32),
            pltpu.VMEM((F, NJ, 1, C), jnp.float32),
            pltpu.VMEM((_LANES,), jnp.float32),
            pltpu.VMEM((bpw,), jnp.float32),
            pltpu.SemaphoreType.DMA,
        ],
    )
    def k(xt_hbm, tab_hbm, bias_hbm, out_hbm, xt_v, vals_v, bias_v, acc_v, gsem):
        wid = lax.axis_index("s") * _NUM_CORES + lax.axis_index("c")
        base = wid * bpw
        pltpu.sync_copy(xt_hbm.at[:, wid], xt_v)
        pltpu.sync_copy(bias_hbm, bias_v)
        copies = [
            pltpu.async_copy(tab_hbm.at[xt_v.at[f, j]], vals_v.at[f, j], gsem)
            for f in range(F)
            for j in range(NJ)
        ]
        for c in copies:
            c.wait()
        bvec = bias_v[...]
        per_chunk = C // _LANES
        for i in range(bpw // _LANES):
            j, off = i // per_chunk, (i % per_chunk) * _LANES
            acc = bvec
            for f in range(F):
                acc = acc + vals_v[f, j, 0, pl.ds(off, _LANES)]
            acc_v[pl.ds(i * _LANES, _LANES)] = acc
        pltpu.sync_copy(acc_v, out_hbm.at[pl.ds(base, bpw)])

    return k(xt, table, bias16)


def kernel(X, table, bias):
    B, F = X.shape
    bpw = B // _NUM_WORKERS
    xt = X.T.reshape(F, _NUM_WORKERS, bpw // _CHUNK, 1, _CHUNK)
    out = _lr_pooled_lookup(xt, table.reshape(1, -1), jnp.broadcast_to(bias, (_LANES,)))
    return out.reshape(B, 1)


# speedup vs baseline: 2.3662x; 1.0908x over previous
import functools

import jax
import jax.numpy as jnp
from jax import lax
from jax.experimental import pallas as pl
from jax.experimental.pallas import tpu as pltpu
from jax.experimental.pallas import tpu_sc as plsc

_NUM_CORES = 2
_NUM_SUBCORES = 16
_NUM_WORKERS = _NUM_CORES * _NUM_SUBCORES
_LANES = 16
_CHUNK = 128


@functools.partial(jax.jit, static_argnums=(3,))
def _lr_pooled_lookup(xt, table, bias16, F):
    NW, _one, L = xt.shape
    C = _CHUNK
    bpw = L // F
    NJ = bpw // C
    B = NW * bpw
    mesh = plsc.VectorSubcoreMesh(core_axis_name="c", subcore_axis_name="s")

    @functools.partial(
        pl.kernel,
        out_type=jax.ShapeDtypeStruct((B,), jnp.float32),
        mesh=mesh,
        scratch_types=[
            pltpu.VMEM((1, 1, L), jnp.int32),
            pltpu.VMEM((1, 1, L), jnp.float32),
            pltpu.VMEM((_LANES,), jnp.float32),
            pltpu.VMEM((bpw,), jnp.float32),
            pltpu.SemaphoreType.DMA,
        ],
    )
    def k(xt_hbm, tab_hbm, bias_hbm, out_hbm, xt_v, vals_v, bias_v, acc_v, gsem):
        wid = lax.axis_index("s") * _NUM_CORES + lax.axis_index("c")
        base = wid * bpw
        pltpu.sync_copy(xt_hbm.at[wid], xt_v.at[0])
        pltpu.sync_copy(bias_hbm, bias_v)
        pltpu.async_copy(tab_hbm.at[xt_v.at[0]], vals_v.at[0], gsem).wait()
        bvec = bias_v[...]
        per_chunk = C // _LANES
        for i in range(bpw // _LANES):
            j, off = i // per_chunk, (i % per_chunk) * _LANES
            acc = bvec
            for f in range(F):
                acc = acc + vals_v[0, 0, pl.ds((f * NJ + j) * C + off, _LANES)]
            acc_v[pl.ds(i * _LANES, _LANES)] = acc
        pltpu.sync_copy(acc_v, out_hbm.at[pl.ds(base, bpw)])

    return k(xt, table, bias16)


def kernel(X, table, bias):
    B, F = X.shape
    bpw = B // _NUM_WORKERS
    NJ = bpw // _CHUNK
    xt = (
        X.T.reshape(F, _NUM_WORKERS, NJ, _CHUNK)
        .swapaxes(0, 1)
        .reshape(_NUM_WORKERS, 1, F * NJ * _CHUNK)
    )
    out = _lr_pooled_lookup(xt, table.reshape(1, -1), jnp.broadcast_to(bias, (_LANES,)), F)
    return out.reshape(B, 1)


# speedup vs baseline: 2.3729x; 1.0028x over previous
import functools

import jax
import jax.numpy as jnp
from jax import lax
from jax.experimental import pallas as pl
from jax.experimental.pallas import tpu as pltpu
from jax.experimental.pallas import tpu_sc as plsc

_NUM_CORES = 2
_NUM_SUBCORES = 16
_NUM_WORKERS = _NUM_CORES * _NUM_SUBCORES
_LANES = 16
_CHUNK = 128


@jax.jit
def _lr_pooled_lookup(xt, table, bias16):
    F, NW, bpw = xt.shape
    C = _CHUNK
    NJ = bpw // C
    L = F * bpw
    B = NW * bpw
    mesh = plsc.VectorSubcoreMesh(core_axis_name="c", subcore_axis_name="s")

    @functools.partial(
        pl.kernel,
        out_type=jax.ShapeDtypeStruct((B,), jnp.float32),
        mesh=mesh,
        scratch_types=[
            pltpu.VMEM((1, 1, L), jnp.int32),
            pltpu.VMEM((1, 1, L), jnp.float32),
            pltpu.VMEM((_LANES,), jnp.float32),
            pltpu.VMEM((bpw,), jnp.float32),
            pltpu.SemaphoreType.DMA,
            pltpu.SemaphoreType.DMA,
        ],
    )
    def k(xt_hbm, tab_hbm, bias_hbm, out_hbm, xt_v, vals_v, bias_v, acc_v, gsem, ssem):
        wid = lax.axis_index("s") * _NUM_CORES + lax.axis_index("c")
        base = wid * bpw
        stages = [
            pltpu.async_copy(
                xt_hbm.at[f, wid], xt_v.at[0, 0, pl.ds(f * bpw, bpw)], ssem
            )
            for f in range(F)
        ]
        pltpu.sync_copy(bias_hbm, bias_v)
        for s in stages:
            s.wait()
        pltpu.async_copy(tab_hbm.at[xt_v.at[0]], vals_v.at[0], gsem).wait()
        bvec = bias_v[...]
        per_chunk = C // _LANES
        for i in range(bpw // _LANES):
            j, off = i // per_chunk, (i % per_chunk) * _LANES
            acc = bvec
            for f in range(F):
                acc = acc + vals_v[0, 0, pl.ds((f * NJ + j) * C + off, _LANES)]
            acc_v[pl.ds(i * _LANES, _LANES)] = acc
        pltpu.sync_copy(acc_v, out_hbm.at[pl.ds(base, bpw)])

    return k(xt, table, bias16)


def kernel(X, table, bias):
    B, F = X.shape
    bpw = B // _NUM_WORKERS
    NJ = bpw // _CHUNK
    xt = X.T.reshape(F, _NUM_WORKERS, bpw)
    out = _lr_pooled_lookup(xt, table.reshape(1, -1), jnp.broadcast_to(bias, (_LANES,)))
    return out.reshape(B, 1)
